# baseline (speedup 1.0000x reference)
import functools

import jax
import jax.numpy as jnp
import numpy as np
from jax.experimental import pallas as pl
from jax.experimental.pallas import tpu as pltpu

_TILE = 408


def _fold_kernel(cw2d_ref, wd_ref, cb_ref, bd_ref, perm_ref,
                 w2_ref, bias_ref, scr_ref):
    wd = wd_ref[...]
    d = wd.shape[0]
    kc_w = 576
    for kc in range(0, 3 * d, kc_w):
        part = jax.lax.dot_general(
            cw2d_ref[:, kc:kc + kc_w], wd, (((0,), (0,)), ((), ())),
            preferred_element_type=jnp.float32)
        scr_ref[kc:kc + kc_w, :] = part.astype(jnp.bfloat16)
    for t in range(3):
        w2_ref[t * d:(t + 1) * d] = jnp.dot(
            perm_ref[t], scr_ref[...], preferred_element_type=jnp.float32
        ).astype(jnp.bfloat16)
    bias_ref[...] = (
        4.0 * jnp.dot(cb_ref[...], wd, preferred_element_type=jnp.float32)
        + bd_ref[...]
    )


def _main_kernel(x_ref, w2_ref, q_ref, bias_ref, o_ref, scr_ref,
                 *, tile, n_tiles, length, d):
    lt = pl.program_id(1)
    start = lt * tile

    xc = x_ref[0, pl.ds(start, tile), :]
    prev_start = pl.multiple_of(jnp.maximum(start - 8, 0), 8)
    next_start = pl.multiple_of(jnp.minimum(start + tile, length - 8), 8)
    prev = x_ref[0, pl.ds(prev_start, 8), :][7:8]
    nxt = x_ref[0, pl.ds(next_start, 8), :][0:1]
    prev = jnp.where(lt == 0, 0.0, prev)
    nxt = jnp.where(lt == n_tiles - 1, 0.0, nxt)

    xwin = jnp.concatenate([prev, xc, nxt], axis=0).astype(jnp.bfloat16)

    d0 = jnp.dot(xwin, w2_ref[0:d], preferred_element_type=jnp.float32)
    d1 = jnp.dot(xwin, w2_ref[d:2 * d], preferred_element_type=jnp.float32)
    d2 = jnp.dot(xwin, w2_ref[2 * d:3 * d], preferred_element_type=jnp.float32)
    scr_ref[0, 0:tile + 2, :] = d1
    scr_ref[1, 0:tile + 2, :] = d2
    a = (d0[0:tile]
         + scr_ref[0, pl.ds(1, tile), :]
         + scr_ref[1, pl.ds(2, tile), :])
    s = jnp.dot(q_ref[...], a.astype(jnp.bfloat16),
                preferred_element_type=jnp.float32)
    o_ref[0] = s + bias_ref[...]


@functools.lru_cache(maxsize=None)
def _pool_matrix(tile):
    q = np.zeros((tile, tile), np.float32)
    for b in (1, 2, 3, 4):
        q += np.kron(np.eye(tile // b, dtype=np.float32),
                     np.full((b, b), 1.0 / b, np.float32))
    return q


@functools.lru_cache(maxsize=None)
def _tap_perm(d):
    p = np.zeros((3, d, 3 * d), np.float32)
    for t in range(3):
        for i in range(d):
            p[t, i, 3 * i + t] = 1.0
    return p


def kernel(X, conv_w, conv_b, wr, br, wd, bd):
    n, length, d = X.shape
    del wr, br
    tile = _TILE
    n_tiles = length // tile

    cw2d = conv_w.reshape(d, 3 * d)
    perm = jnp.asarray(_tap_perm(d), dtype=jnp.bfloat16)
    w2, bias = pl.pallas_call(
        _fold_kernel,
        out_shape=(
            jax.ShapeDtypeStruct((3 * d, d), jnp.bfloat16),
            jax.ShapeDtypeStruct((1, d), jnp.float32),
        ),
        scratch_shapes=[pltpu.VMEM((3 * d, d), jnp.bfloat16)],
        compiler_params=pltpu.CompilerParams(
            vmem_limit_bytes=50 * 1024 * 1024,
        ),
        name="gbst_fold",
    )(cw2d, wd, conv_b.reshape(1, d), bd.reshape(1, d), perm)

    q = jnp.asarray(_pool_matrix(tile), dtype=jnp.bfloat16)

    body = functools.partial(
        _main_kernel, tile=tile, n_tiles=n_tiles, length=length, d=d)
    out = pl.pallas_call(
        body,
        grid=(n, n_tiles),
        in_specs=[
            pl.BlockSpec((1, length, d), lambda i, j: (i, 0, 0)),
            pl.BlockSpec((3 * d, d), lambda i, j: (0, 0)),
            pl.BlockSpec((tile, tile), lambda i, j: (0, 0)),
            pl.BlockSpec((1, d), lambda i, j: (0, 0)),
        ],
        out_specs=pl.BlockSpec((1, tile, d), lambda i, j: (i, j, 0)),
        out_shape=jax.ShapeDtypeStruct((n, length, d), jnp.float32),
        scratch_shapes=[pltpu.VMEM((2, tile + 8, d), jnp.float32)],
        compiler_params=pltpu.CompilerParams(
            dimension_semantics=("parallel", "arbitrary"),
            vmem_limit_bytes=50 * 1024 * 1024,
        ),
        name="gbst_main",
    )(X, w2, q, bias)
    return out


# speedup vs baseline: 1.4738x; 1.4738x over previous
import functools

import jax
import jax.numpy as jnp
import numpy as np
from jax.experimental import pallas as pl
from jax.experimental.pallas import tpu as pltpu

_TILE = 408


def _fold_kernel(cwt_ref, wd_ref, cb_ref, bd_ref, w2_ref, bias_ref):
    wd = wd_ref[...]
    for t in range(3):
        w2_ref[t] = jax.lax.dot_general(
            cwt_ref[t], wd, (((0,), (0,)), ((), ())),
            preferred_element_type=jnp.float32
        ).astype(jnp.bfloat16)
    bias_ref[...] = (
        4.0 * jnp.dot(cb_ref[...], wd, preferred_element_type=jnp.float32)
        + bd_ref[...]
    )


def _main_kernel(x_ref, w2_ref, q_ref, bias_ref, o_ref, scr_ref,
                 *, tile, n_tiles, length):
    lt = pl.program_id(1)
    start = lt * tile

    xc = x_ref[0, pl.ds(start, tile), :]
    prev_start = pl.multiple_of(jnp.maximum(start - 8, 0), 8)
    next_start = pl.multiple_of(jnp.minimum(start + tile, length - 8), 8)
    prev = x_ref[0, pl.ds(prev_start, 8), :][7:8]
    nxt = x_ref[0, pl.ds(next_start, 8), :][0:1]
    prev = jnp.where(lt == 0, 0.0, prev)
    nxt = jnp.where(lt == n_tiles - 1, 0.0, nxt)

    xwin = jnp.concatenate([prev, xc, nxt], axis=0).astype(jnp.bfloat16)

    d0 = jnp.dot(xwin, w2_ref[0], preferred_element_type=jnp.float32)
    d1 = jnp.dot(xwin, w2_ref[1], preferred_element_type=jnp.float32)
    d2 = jnp.dot(xwin, w2_ref[2], preferred_element_type=jnp.float32)
    scr_ref[0, 0:tile + 2, :] = d1
    scr_ref[1, 0:tile + 2, :] = d2
    a = (d0[0:tile]
         + scr_ref[0, pl.ds(1, tile), :]
         + scr_ref[1, pl.ds(2, tile), :])
    s = jnp.dot(q_ref[...], a.astype(jnp.bfloat16),
                preferred_element_type=jnp.float32)
    o_ref[0] = s + bias_ref[...]


@functools.lru_cache(maxsize=None)
def _pool_matrix(tile):
    q = np.zeros((tile, tile), np.float32)
    for b in (1, 2, 3, 4):
        q += np.kron(np.eye(tile // b, dtype=np.float32),
                     np.full((b, b), 1.0 / b, np.float32))
    return q


def kernel(X, conv_w, conv_b, wr, br, wd, bd):
    n, length, d = X.shape
    del wr, br
    tile = _TILE
    n_tiles = length // tile

    cwt = conv_w.transpose(2, 0, 1)
    w2, bias = pl.pallas_call(
        _fold_kernel,
        out_shape=(
            jax.ShapeDtypeStruct((3, d, d), jnp.bfloat16),
            jax.ShapeDtypeStruct((1, d), jnp.float32),
        ),
        name="gbst_fold",
    )(cwt, wd, conv_b.reshape(1, d), bd.reshape(1, d))

    q = jnp.asarray(_pool_matrix(tile), dtype=jnp.bfloat16)

    body = functools.partial(
        _main_kernel, tile=tile, n_tiles=n_tiles, length=length)
    out = pl.pallas_call(
        body,
        grid=(n, n_tiles),
        in_specs=[
            pl.BlockSpec((1, length, d), lambda i, j: (i, 0, 0)),
            pl.BlockSpec((3, d, d), lambda i, j: (0, 0, 0)),
            pl.BlockSpec((tile, tile), lambda i, j: (0, 0)),
            pl.BlockSpec((1, d), lambda i, j: (0, 0)),
        ],
        out_specs=pl.BlockSpec((1, tile, d), lambda i, j: (i, j, 0)),
        out_shape=jax.ShapeDtypeStruct((n, length, d), jnp.float32),
        scratch_shapes=[pltpu.VMEM((2, tile + 8, d), jnp.float32)],
        compiler_params=pltpu.CompilerParams(
            dimension_semantics=("parallel", "arbitrary"),
            vmem_limit_bytes=50 * 1024 * 1024,
        ),
        name="gbst_main",
    )(X, w2, q, bias)
    return out


# speedup vs baseline: 1.6146x; 1.0955x over previous
import functools

import jax
import jax.numpy as jnp
import numpy as np
from jax.experimental import pallas as pl
from jax.experimental.pallas import tpu as pltpu

_TILE = 408


def _fold_kernel(cwt_ref, wd_ref, cb_ref, bd_ref, w2_ref, bias_ref):
    wd = wd_ref[...]
    for t in range(3):
        w2_ref[t] = jax.lax.dot_general(
            cwt_ref[t], wd, (((0,), (0,)), ((), ())),
            preferred_element_type=jnp.float32
        ).astype(jnp.bfloat16)
    bias_ref[...] = (
        4.0 * jnp.dot(cb_ref[...], wd, preferred_element_type=jnp.float32)
        + bd_ref[...]
    )


def _main_kernel(xc_ref, ph_ref, nh_ref, w2_ref, q_ref, bias_ref, o_ref,
                 scr_ref, *, tile, n_tiles, length):
    lt = pl.program_id(1)

    xc = xc_ref[0]
    prev = jnp.where(lt == 0, 0.0, ph_ref[0, 7:8, :])
    nxt = jnp.where(lt == n_tiles - 1, 0.0, nh_ref[0, 0:1, :])

    xwin = jnp.concatenate([prev, xc, nxt], axis=0).astype(jnp.bfloat16)

    d0 = jnp.dot(xwin, w2_ref[0], preferred_element_type=jnp.float32)
    d1 = jnp.dot(xwin, w2_ref[1], preferred_element_type=jnp.float32)
    d2 = jnp.dot(xwin, w2_ref[2], preferred_element_type=jnp.float32)
    scr_ref[0, 0:tile + 2, :] = d1
    scr_ref[1, 0:tile + 2, :] = d2
    a = (d0[0:tile]
         + scr_ref[0, pl.ds(1, tile), :]
         + scr_ref[1, pl.ds(2, tile), :])
    s = jnp.dot(q_ref[...], a.astype(jnp.bfloat16),
                preferred_element_type=jnp.float32)
    o_ref[0] = s + bias_ref[...]


@functools.lru_cache(maxsize=None)
def _pool_matrix(tile):
    q = np.zeros((tile, tile), np.float32)
    for b in (1, 2, 3, 4):
        q += np.kron(np.eye(tile // b, dtype=np.float32),
                     np.full((b, b), 1.0 / b, np.float32))
    return q


def kernel(X, conv_w, conv_b, wr, br, wd, bd):
    n, length, d = X.shape
    del wr, br
    tile = _TILE
    n_tiles = length // tile

    cwt = conv_w.transpose(2, 0, 1)
    w2, bias = pl.pallas_call(
        _fold_kernel,
        out_shape=(
            jax.ShapeDtypeStruct((3, d, d), jnp.bfloat16),
            jax.ShapeDtypeStruct((1, d), jnp.float32),
        ),
        name="gbst_fold",
    )(cwt, wd, conv_b.reshape(1, d), bd.reshape(1, d))

    q = jnp.asarray(_pool_matrix(tile), dtype=jnp.bfloat16)

    body = functools.partial(
        _main_kernel, tile=tile, n_tiles=n_tiles, length=length)
    out = pl.pallas_call(
        body,
        grid=(n, n_tiles),
        in_specs=[
            pl.BlockSpec((1, tile, d), lambda i, j: (i, j, 0)),
            pl.BlockSpec(
                (1, 8, d),
                lambda i, j: (i, jnp.maximum(j * (tile // 8) - 1, 0), 0)),
            pl.BlockSpec(
                (1, 8, d),
                lambda i, j: (i, jnp.minimum((j + 1) * (tile // 8),
                                             length // 8 - 1), 0)),
            pl.BlockSpec((3, d, d), lambda i, j: (0, 0, 0)),
            pl.BlockSpec((tile, tile), lambda i, j: (0, 0)),
            pl.BlockSpec((1, d), lambda i, j: (0, 0)),
        ],
        out_specs=pl.BlockSpec((1, tile, d), lambda i, j: (i, j, 0)),
        out_shape=jax.ShapeDtypeStruct((n, length, d), jnp.float32),
        scratch_shapes=[pltpu.VMEM((2, tile + 8, d), jnp.float32)],
        compiler_params=pltpu.CompilerParams(
            dimension_semantics=("parallel", "arbitrary"),
            vmem_limit_bytes=50 * 1024 * 1024,
        ),
        name="gbst_main",
    )(X, X, X, w2, q, bias)
    return out
